# baseline (speedup 1.0000x reference)
import jax
import jax.numpy as jnp
from jax import lax
from jax.experimental import pallas as pl
from jax.experimental.pallas import tpu as pltpu

B = 16384
L = 200
RBLK = 1024
GRID = B // RBLK


def _tc_kernel(node_ref, items_ref, out_ref):
    node = node_ref[...]
    items = items_ref[...]
    col = lax.broadcasted_iota(jnp.int32, (RBLK, L), 1)
    pick = col == node - 1
    sel = jnp.where(pick, items, jnp.int32(0)).astype(jnp.float32)
    ones = jnp.ones((L, 1), jnp.float32)
    v = jax.lax.dot_general(sel, ones, (((1,), (0,)), ((), ())),
                            preferred_element_type=jnp.float32)
    vi = v.astype(jnp.int32)
    out_ref[...] = jnp.where(node != 0, vi, jnp.int32(-1))


@jax.jit
def _decode(node, items):
    out = pl.pallas_call(
        _tc_kernel,
        grid=(GRID,),
        in_specs=[
            pl.BlockSpec((RBLK, 1), lambda i: (i, 0)),
            pl.BlockSpec((RBLK, L), lambda i: (i, 0)),
        ],
        out_specs=pl.BlockSpec((RBLK, 1), lambda i: (i, 0)),
        out_shape=jax.ShapeDtypeStruct((B, 1), jnp.int32),
        compiler_params=pltpu.CompilerParams(
            dimension_semantics=("arbitrary",),
        ),
    )(node, items)
    return jnp.reshape(out, (B,))


def kernel(x_dummy, current_node, item_ids):
    node = current_node.astype(jnp.int32)
    return _decode(node, item_ids.astype(jnp.int32)).astype(item_ids.dtype)
